# baseline (speedup 1.0000x reference)
import math

import jax
import jax.numpy as jnp
from jax.experimental import pallas as pl
from jax.experimental.pallas import tpu as pltpu

TEMP = 0.05
LOG2E = math.log2(math.e)
LN2 = math.log(2.0)

B = 1024
D = 256
K = 8192
KBLK = 4096
NSTEPS = K // KBLK
LANES = 128


def _lane_sum(a):
    n = a.shape[1] // LANES
    parts = [a[:, g * LANES:(g + 1) * LANES] for g in range(n)]
    while len(parts) > 1:
        parts = [parts[i] + parts[i + 1] for i in range(0, len(parts), 2)]
    return parts[0]


def _loss_kernel(x_ref, t_ref, f_ref, out_ref, xs_ref, acc_ref, tgt_ref):
    k = pl.program_id(0)

    @pl.when(k == 0)
    def _init():
        x = x_ref[...]
        norm = jnp.sqrt(jnp.sum(x * x, axis=1, keepdims=True))
        scale = LOG2E / (jnp.maximum(norm, 1e-12) * TEMP)
        xs_ref[...] = (x * scale).astype(jnp.bfloat16)
        acc_ref[...] = jnp.zeros_like(acc_ref)
        tgt_ref[...] = jnp.zeros_like(tgt_ref)

    s2 = jax.lax.dot_general(
        xs_ref[...], f_ref[...].astype(jnp.bfloat16),
        dimension_numbers=(((1,), (1,)), ((), ())),
        preferred_element_type=jnp.float32,
    )
    acc_ref[...] += _lane_sum(jnp.exp2(s2))

    cols = k * KBLK + jax.lax.broadcasted_iota(jnp.int32, (B, KBLK), 1)
    tgt_ref[...] += _lane_sum(jnp.where(cols == t_ref[...], s2, 0.0))

    @pl.when(k == NSTEPS - 1)
    def _fini():
        lse = jnp.log(jnp.sum(acc_ref[...], axis=1, keepdims=True))
        tgt = jnp.sum(tgt_ref[...], axis=1, keepdims=True) * LN2
        out_ref[...] = jnp.mean(lse - tgt, keepdims=True).reshape(1, 1)


@jax.jit
def _run(inputs, targets, features):
    t2d = targets.astype(jnp.int32).reshape(B, 1)
    out = pl.pallas_call(
        _loss_kernel,
        grid=(NSTEPS,),
        in_specs=[
            pl.BlockSpec((B, D), lambda k: (0, 0)),
            pl.BlockSpec((B, 1), lambda k: (0, 0)),
            pl.BlockSpec((KBLK, D), lambda k: (k, 0)),
        ],
        out_specs=pl.BlockSpec((1, 1), lambda k: (0, 0)),
        out_shape=jax.ShapeDtypeStruct((1, 1), jnp.float32),
        scratch_shapes=[
            pltpu.VMEM((B, D), jnp.bfloat16),
            pltpu.VMEM((B, LANES), jnp.float32),
            pltpu.VMEM((B, LANES), jnp.float32),
        ],
    )(inputs, t2d, features)
    return out[0, 0]


def kernel(inputs, targets, features):
    return _run(inputs, targets, features)


# speedup vs baseline: 1.0768x; 1.0768x over previous
import math

import jax
import jax.numpy as jnp
from jax.experimental import pallas as pl
from jax.experimental.pallas import tpu as pltpu

TEMP = 0.05
LOG2E = math.log2(math.e)
LN2 = math.log(2.0)

B = 1024
D = 256
K = 8192
KBLK = 2048
NSTEPS = K // KBLK
LANES = 128


def _lane_sum(a):
    n = a.shape[1] // LANES
    parts = [a[:, g * LANES:(g + 1) * LANES] for g in range(n)]
    while len(parts) > 1:
        parts = [parts[i] + parts[i + 1] for i in range(0, len(parts), 2)]
    return parts[0]


def _loss_kernel(x_ref, t_ref, f_ref, out_ref, xs_ref, acc_ref, tgt_ref):
    k = pl.program_id(0)

    @pl.when(k == 0)
    def _init():
        x = x_ref[...]
        norm = jnp.sqrt(jnp.sum(x * x, axis=1, keepdims=True))
        scale = LOG2E / (jnp.maximum(norm, 1e-12) * TEMP)
        xs_ref[...] = (x * scale).astype(jnp.bfloat16)
        acc_ref[...] = jnp.zeros_like(acc_ref)
        tgt_ref[...] = jnp.zeros_like(tgt_ref)

    s2 = jax.lax.dot_general(
        xs_ref[...], f_ref[...].astype(jnp.bfloat16),
        dimension_numbers=(((1,), (1,)), ((), ())),
        preferred_element_type=jnp.float32,
    )
    acc_ref[...] += _lane_sum(jnp.exp2(s2))

    cols = k * KBLK + jax.lax.broadcasted_iota(jnp.int32, (B, KBLK), 1)
    tgt_ref[...] += _lane_sum(jnp.where(cols == t_ref[...], s2, 0.0))

    @pl.when(k == NSTEPS - 1)
    def _fini():
        lse = jnp.log(jnp.sum(acc_ref[...], axis=1, keepdims=True))
        tgt = jnp.sum(tgt_ref[...], axis=1, keepdims=True) * LN2
        out_ref[...] = jnp.mean(lse - tgt, keepdims=True).reshape(1, 1)


@jax.jit
def _run(inputs, targets, features):
    t2d = targets.astype(jnp.int32).reshape(B, 1)
    out = pl.pallas_call(
        _loss_kernel,
        grid=(NSTEPS,),
        in_specs=[
            pl.BlockSpec((B, D), lambda k: (0, 0)),
            pl.BlockSpec((B, 1), lambda k: (0, 0)),
            pl.BlockSpec((KBLK, D), lambda k: (k, 0)),
        ],
        out_specs=pl.BlockSpec((1, 1), lambda k: (0, 0)),
        out_shape=jax.ShapeDtypeStruct((1, 1), jnp.float32),
        scratch_shapes=[
            pltpu.VMEM((B, D), jnp.bfloat16),
            pltpu.VMEM((B, LANES), jnp.float32),
            pltpu.VMEM((B, LANES), jnp.float32),
        ],
    )(inputs, t2d, features)
    return out[0, 0]


def kernel(inputs, targets, features):
    return _run(inputs, targets, features)
